# baseline (speedup 1.0000x reference)
import functools

import jax
import jax.numpy as jnp
from jax import lax
from jax.experimental import pallas as pl
from jax.experimental.pallas import tpu as pltpu
from jax.experimental.pallas import tpu_sc as plsc

B = 1024
D = 2048
C = 1000
CPAD = 1024
K = 16
NUM_DROP = 512
IDX_PAD = 128

SC_CORES = 2
SC_SUBCORES = 16
NW = SC_CORES * SC_SUBCORES
RPW = B // NW
CHUNK = 8


def _topk_idx_body(p_ref, y_ref, out_ref):
    p = p_ref[...]
    rb = p.shape[0]
    col = lax.broadcasted_iota(jnp.int32, (rb, CPAD), 1)
    ocol = lax.broadcasted_iota(jnp.int32, (rb, IDX_PAD), 1)
    acc = jnp.broadcast_to(y_ref[...], (rb, IDX_PAD))
    for k in range(K):
        m = jnp.max(p, axis=1, keepdims=True)
        idx = jnp.min(jnp.where(p == m, col, CPAD), axis=1, keepdims=True)
        acc = jnp.where(ocol == k, idx, acc)
        p = jnp.where(col == idx, -jnp.inf, p)
    out_ref[...] = acc


def _topk_indices(p_pad, y2):
    rb = 128
    return pl.pallas_call(
        _topk_idx_body,
        grid=(B // rb,),
        in_specs=[
            pl.BlockSpec((rb, CPAD), lambda i: (i, 0)),
            pl.BlockSpec((rb, 1), lambda i: (i, 0)),
        ],
        out_specs=pl.BlockSpec((rb, IDX_PAD), lambda i: (i, 0)),
        out_shape=jax.ShapeDtypeStruct((B, IDX_PAD), jnp.int32),
    )(p_pad, y2)


def _make_sc_scores_body(rpw):
  def _sc_scores_body(idx_hbm, y_hbm, w_hbm, scores_hbm,
                      idxbuf, yidx, ybuf, wbuf0, wbuf1, srow0, srow1,
                      sem_i, sem_y, sem_w0, sem_w1, sem_s0, sem_s1):
    wid = lax.axis_index("s") * SC_CORES + lax.axis_index("c")
    base = wid * rpw
    wbufs = (wbuf0, wbuf1)
    sem_ws = (sem_w0, sem_w1)
    srows = (srow0, srow1)
    sem_ss = (sem_s0, sem_s1)

    pltpu.async_copy(idx_hbm.at[pl.ds(base, rpw)], idxbuf, sem_i).wait()
    pltpu.async_copy(y_hbm.at[pl.ds(base, rpw)], yidx, sem_i).wait()

    def gather_w(r, slot):
        pltpu.async_copy(
            w_hbm.at[idxbuf.at[r, pl.ds(0, K)]], wbufs[slot], sem_ws[slot])

    gather_w(0, 0)

    def chunk_body(c, carry):
        pltpu.async_copy(
            w_hbm.at[yidx.at[pl.ds(c * CHUNK, CHUNK)]], ybuf, sem_y).wait()

        for i in range(CHUNK):
            r = c * CHUNK + i
            slot = i % 2
            wbuf = wbufs[slot]
            pltpu.make_async_copy(
                w_hbm.at[idxbuf.at[r, pl.ds(0, K)]], wbuf, sem_ws[slot]).wait()
            if i < CHUNK - 1:
                gather_w(r + 1, 1 - slot)
            else:
                @pl.when(c < rpw // CHUNK - 1)
                def _():
                    gather_w(r + 1, 1 - slot)

            srow = srows[slot]
            if i >= 2:
                pltpu.make_async_copy(
                    srow, scores_hbm.at[base + r - 2], sem_ss[slot]).wait()
            else:
                @pl.when(c > 0)
                def _():
                    pltpu.make_async_copy(
                        srow, scores_hbm.at[base + r - 2], sem_ss[slot]).wait()

            def col_body(j, carry2, wbuf=wbuf, srow=srow, i=i):
                sl = pl.ds(j * 16, 16)
                hi = wbuf[0, sl]
                lo = hi
                for k in range(1, K):
                    v = wbuf[k, sl]
                    hi = jnp.maximum(hi, v)
                    lo = jnp.minimum(lo, v)
                wy = ybuf[i, sl]
                srow[sl] = jnp.maximum(hi - wy, wy - lo)
                return carry2

            lax.fori_loop(0, D // 16, col_body, 0)
            pltpu.async_copy(srow, scores_hbm.at[base + r], sem_ss[slot])
        return carry

    lax.fori_loop(0, rpw // CHUNK, chunk_body, 0)
    for i in range(2):
        pltpu.make_async_copy(
            srows[i], scores_hbm.at[base + rpw - 2 + i], sem_ss[i]).wait()


  return _sc_scores_body


@functools.cache
def _sc_scores_kernel(nb):
    rpw = nb // NW
    mesh = plsc.VectorSubcoreMesh(
        core_axis_name="c", subcore_axis_name="s",
        num_cores=SC_CORES, num_subcores=SC_SUBCORES)
    return pl.kernel(
        _make_sc_scores_body(rpw),
        out_type=jax.ShapeDtypeStruct((nb, D), jnp.float32),
        mesh=mesh,
        scratch_types=[
            pltpu.VMEM((rpw, IDX_PAD), jnp.int32),
            pltpu.VMEM((rpw,), jnp.int32),
            pltpu.VMEM((CHUNK, D), jnp.float32),
            pltpu.VMEM((K, D), jnp.float32),
            pltpu.VMEM((K, D), jnp.float32),
            pltpu.VMEM((D,), jnp.float32),
            pltpu.VMEM((D,), jnp.float32),
            pltpu.SemaphoreType.DMA,
            pltpu.SemaphoreType.DMA,
            pltpu.SemaphoreType.DMA,
            pltpu.SemaphoreType.DMA,
            pltpu.SemaphoreType.DMA,
            pltpu.SemaphoreType.DMA,
        ],
    )


def _select_body(s_ref, x_ref, o_ref):
    bits = lax.bitcast_convert_type(s_ref[...], jnp.int32)
    rb = bits.shape[0]

    def round_body(t, prefix):
        cand = prefix | (1 << (30 - t))
        cnt = jnp.sum((bits >= cand).astype(jnp.int32), axis=1, keepdims=True)
        return jnp.where(cnt >= NUM_DROP, cand, prefix)

    prefix = lax.fori_loop(0, 31, round_body, jnp.zeros((rb, 1), jnp.int32))
    o_ref[...] = jnp.where(bits >= prefix, 0.0, x_ref[...])


def _select_mask(scores, x):
    rb = 256
    nb = scores.shape[0]
    return pl.pallas_call(
        _select_body,
        grid=(nb // rb,),
        in_specs=[
            pl.BlockSpec((rb, D), lambda i: (i, 0)),
            pl.BlockSpec((rb, D), lambda i: (i, 0)),
        ],
        out_specs=pl.BlockSpec((rb, D), lambda i: (i, 0)),
        out_shape=jax.ShapeDtypeStruct((nb, D), jnp.float32),
    )(scores, x)


def kernel(x, y, weight_matrix, prev_output):
    y1 = y.astype(jnp.int32)
    p_pad = jnp.pad(prev_output, ((0, 0), (0, CPAD - C)),
                    constant_values=-jnp.inf)
    idx = _topk_indices(p_pad, y1.reshape(B, 1))
    half = B // 2
    outs = []
    for h in range(2):
        sl = slice(h * half, (h + 1) * half)
        s = _sc_scores_kernel(half)(idx[sl], y1[sl], weight_matrix)
        outs.append(_select_mask(s, x[sl]))
    return jnp.concatenate(outs, axis=0)


# speedup vs baseline: 1.0879x; 1.0879x over previous
import functools

import jax
import jax.numpy as jnp
from jax import lax
from jax.experimental import pallas as pl
from jax.experimental.pallas import tpu as pltpu
from jax.experimental.pallas import tpu_sc as plsc

B = 1024
D = 2048
C = 1000
CPAD = 1024
K = 16
NUM_DROP = 512
IDX_PAD = 128

SC_CORES = 2
SC_SUBCORES = 16
NW = SC_CORES * SC_SUBCORES
RPW = B // NW
CHUNK = 8


def _topk_idx_body(p_ref, y_ref, out_ref):
    p = p_ref[...]
    rb = p.shape[0]
    col = lax.broadcasted_iota(jnp.int32, (rb, CPAD), 1)
    ocol = lax.broadcasted_iota(jnp.int32, (rb, IDX_PAD), 1)
    acc = jnp.broadcast_to(y_ref[...], (rb, IDX_PAD))
    for k in range(K):
        m = jnp.max(p, axis=1, keepdims=True)
        idx = jnp.min(jnp.where(p == m, col, CPAD), axis=1, keepdims=True)
        acc = jnp.where(ocol == k, idx, acc)
        p = jnp.where(col == idx, -jnp.inf, p)
    out_ref[...] = acc


def _topk_indices(p_pad, y2):
    rb = 256
    return pl.pallas_call(
        _topk_idx_body,
        grid=(B // rb,),
        in_specs=[
            pl.BlockSpec((rb, CPAD), lambda i: (i, 0)),
            pl.BlockSpec((rb, 1), lambda i: (i, 0)),
        ],
        out_specs=pl.BlockSpec((rb, IDX_PAD), lambda i: (i, 0)),
        out_shape=jax.ShapeDtypeStruct((B, IDX_PAD), jnp.int32),
    )(p_pad, y2)


def _make_sc_scores_body(rpw):
  def _sc_scores_body(idx_hbm, y_hbm, w_hbm, scores_hbm,
                      idxbuf, yidx, ybuf, wbuf0, wbuf1, srow0, srow1,
                      sem_i, sem_y, sem_w0, sem_w1, sem_s0, sem_s1):
    wid = lax.axis_index("s") * SC_CORES + lax.axis_index("c")
    base = wid * rpw
    wbufs = (wbuf0, wbuf1)
    sem_ws = (sem_w0, sem_w1)
    srows = (srow0, srow1)
    sem_ss = (sem_s0, sem_s1)

    pltpu.async_copy(idx_hbm.at[pl.ds(base, rpw)], idxbuf, sem_i).wait()
    pltpu.async_copy(y_hbm.at[pl.ds(base, rpw)], yidx, sem_i).wait()

    def gather_w(r, slot):
        pltpu.async_copy(
            w_hbm.at[idxbuf.at[r, pl.ds(0, K)]], wbufs[slot], sem_ws[slot])

    gather_w(0, 0)

    def chunk_body(c, carry):
        pltpu.async_copy(
            w_hbm.at[yidx.at[pl.ds(c * CHUNK, CHUNK)]], ybuf, sem_y).wait()

        for i in range(CHUNK):
            r = c * CHUNK + i
            slot = i % 2
            wbuf = wbufs[slot]
            pltpu.make_async_copy(
                w_hbm.at[idxbuf.at[r, pl.ds(0, K)]], wbuf, sem_ws[slot]).wait()
            if i < CHUNK - 1:
                gather_w(r + 1, 1 - slot)
            else:
                @pl.when(c < rpw // CHUNK - 1)
                def _():
                    gather_w(r + 1, 1 - slot)

            srow = srows[slot]
            if i >= 2:
                pltpu.make_async_copy(
                    srow, scores_hbm.at[base + r - 2], sem_ss[slot]).wait()
            else:
                @pl.when(c > 0)
                def _():
                    pltpu.make_async_copy(
                        srow, scores_hbm.at[base + r - 2], sem_ss[slot]).wait()

            def col_body(j, carry2, wbuf=wbuf, srow=srow, i=i):
                sl = pl.ds(j * 16, 16)
                hi = wbuf[0, sl]
                lo = hi
                for k in range(1, K):
                    v = wbuf[k, sl]
                    hi = jnp.maximum(hi, v)
                    lo = jnp.minimum(lo, v)
                wy = ybuf[i, sl]
                srow[sl] = jnp.maximum(hi - wy, wy - lo)
                return carry2

            lax.fori_loop(0, D // 16, col_body, 0)
            pltpu.async_copy(srow, scores_hbm.at[base + r], sem_ss[slot])
        return carry

    lax.fori_loop(0, rpw // CHUNK, chunk_body, 0)
    for i in range(2):
        pltpu.make_async_copy(
            srows[i], scores_hbm.at[base + rpw - 2 + i], sem_ss[i]).wait()


  return _sc_scores_body


@functools.cache
def _sc_scores_kernel(nb):
    rpw = nb // NW
    mesh = plsc.VectorSubcoreMesh(
        core_axis_name="c", subcore_axis_name="s",
        num_cores=SC_CORES, num_subcores=SC_SUBCORES)
    return pl.kernel(
        _make_sc_scores_body(rpw),
        out_type=jax.ShapeDtypeStruct((nb, D), jnp.float32),
        mesh=mesh,
        scratch_types=[
            pltpu.VMEM((rpw, IDX_PAD), jnp.int32),
            pltpu.VMEM((rpw,), jnp.int32),
            pltpu.VMEM((CHUNK, D), jnp.float32),
            pltpu.VMEM((K, D), jnp.float32),
            pltpu.VMEM((K, D), jnp.float32),
            pltpu.VMEM((D,), jnp.float32),
            pltpu.VMEM((D,), jnp.float32),
            pltpu.SemaphoreType.DMA,
            pltpu.SemaphoreType.DMA,
            pltpu.SemaphoreType.DMA,
            pltpu.SemaphoreType.DMA,
            pltpu.SemaphoreType.DMA,
            pltpu.SemaphoreType.DMA,
        ],
    )


def _select_body(s_ref, x_ref, o_ref):
    bits = lax.bitcast_convert_type(s_ref[...], jnp.int32)
    rb = bits.shape[0]

    def round_body(t, prefix):
        cand = prefix | (1 << (30 - t))
        cnt = jnp.sum((bits >= cand).astype(jnp.int32), axis=1, keepdims=True)
        return jnp.where(cnt >= NUM_DROP, cand, prefix)

    prefix = lax.fori_loop(0, 31, round_body, jnp.zeros((rb, 1), jnp.int32))
    o_ref[...] = jnp.where(bits >= prefix, 0.0, x_ref[...])


def _select_mask(scores, x):
    rb = 512
    nb = scores.shape[0]
    return pl.pallas_call(
        _select_body,
        grid=(nb // rb,),
        in_specs=[
            pl.BlockSpec((rb, D), lambda i: (i, 0)),
            pl.BlockSpec((rb, D), lambda i: (i, 0)),
        ],
        out_specs=pl.BlockSpec((rb, D), lambda i: (i, 0)),
        out_shape=jax.ShapeDtypeStruct((nb, D), jnp.float32),
    )(scores, x)


def kernel(x, y, weight_matrix, prev_output):
    y1 = y.astype(jnp.int32)
    p_pad = jnp.pad(prev_output, ((0, 0), (0, CPAD - C)),
                    constant_values=-jnp.inf)
    idx = _topk_indices(p_pad, y1.reshape(B, 1))
    scores = _sc_scores_kernel(B)(idx, y1, weight_matrix)
    return _select_mask(scores, x)


# speedup vs baseline: 1.0978x; 1.0091x over previous
import functools

import jax
import jax.numpy as jnp
from jax import lax
from jax.experimental import pallas as pl
from jax.experimental.pallas import tpu as pltpu
from jax.experimental.pallas import tpu_sc as plsc

B = 1024
D = 2048
C = 1000
CPAD = 1024
K = 16
NUM_DROP = 512
IDX_PAD = 128

SC_CORES = 2
SC_SUBCORES = 16
NW = SC_CORES * SC_SUBCORES
RPW = B // NW
CHUNK = 8


def _topk_idx_body(p_ref, y_ref, out_ref):
    p = p_ref[...]
    rb = p.shape[0]
    col = lax.broadcasted_iota(jnp.int32, (rb, CPAD), 1)
    ocol = lax.broadcasted_iota(jnp.int32, (rb, IDX_PAD), 1)
    acc = jnp.broadcast_to(y_ref[...], (rb, IDX_PAD))
    for k in range(K):
        m = jnp.max(p, axis=1, keepdims=True)
        idx = jnp.min(jnp.where(p == m, col, CPAD), axis=1, keepdims=True)
        acc = jnp.where(ocol == k, idx, acc)
        p = jnp.where(col == idx, -jnp.inf, p)
    out_ref[...] = acc


def _topk_indices(p_pad, y2):
    rb = 512
    return pl.pallas_call(
        _topk_idx_body,
        grid=(B // rb,),
        in_specs=[
            pl.BlockSpec((rb, CPAD), lambda i: (i, 0)),
            pl.BlockSpec((rb, 1), lambda i: (i, 0)),
        ],
        out_specs=pl.BlockSpec((rb, IDX_PAD), lambda i: (i, 0)),
        out_shape=jax.ShapeDtypeStruct((B, IDX_PAD), jnp.int32),
    )(p_pad, y2)


def _make_sc_scores_body(rpw):
  def _sc_scores_body(idx_hbm, y_hbm, w_hbm, scores_hbm,
                      idxbuf, yidx, ybuf, wbuf0, wbuf1, srow0, srow1,
                      sem_i, sem_y, sem_w0, sem_w1, sem_s0, sem_s1):
    wid = lax.axis_index("s") * SC_CORES + lax.axis_index("c")
    base = wid * rpw
    wbufs = (wbuf0, wbuf1)
    sem_ws = (sem_w0, sem_w1)
    srows = (srow0, srow1)
    sem_ss = (sem_s0, sem_s1)

    pltpu.async_copy(idx_hbm.at[pl.ds(base, rpw)], idxbuf, sem_i).wait()
    pltpu.async_copy(y_hbm.at[pl.ds(base, rpw)], yidx, sem_i).wait()

    def gather_w(r, slot):
        pltpu.async_copy(
            w_hbm.at[idxbuf.at[r, pl.ds(0, K)]], wbufs[slot], sem_ws[slot])

    gather_w(0, 0)

    def chunk_body(c, carry):
        pltpu.async_copy(
            w_hbm.at[yidx.at[pl.ds(c * CHUNK, CHUNK)]], ybuf, sem_y).wait()

        for i in range(CHUNK):
            r = c * CHUNK + i
            slot = i % 2
            wbuf = wbufs[slot]
            pltpu.make_async_copy(
                w_hbm.at[idxbuf.at[r, pl.ds(0, K)]], wbuf, sem_ws[slot]).wait()
            if i < CHUNK - 1:
                gather_w(r + 1, 1 - slot)
            else:
                @pl.when(c < rpw // CHUNK - 1)
                def _():
                    gather_w(r + 1, 1 - slot)

            srow = srows[slot]
            if i >= 2:
                pltpu.make_async_copy(
                    srow, scores_hbm.at[base + r - 2], sem_ss[slot]).wait()
            else:
                @pl.when(c > 0)
                def _():
                    pltpu.make_async_copy(
                        srow, scores_hbm.at[base + r - 2], sem_ss[slot]).wait()

            def col_body(j, carry2, wbuf=wbuf, srow=srow, i=i):
                sl = pl.ds(j * 16, 16)
                hi = wbuf[0, sl]
                lo = hi
                for k in range(1, K):
                    v = wbuf[k, sl]
                    hi = jnp.maximum(hi, v)
                    lo = jnp.minimum(lo, v)
                wy = ybuf[i, sl]
                srow[sl] = jnp.maximum(hi - wy, wy - lo)
                return carry2

            lax.fori_loop(0, D // 16, col_body, 0)
            pltpu.async_copy(srow, scores_hbm.at[base + r], sem_ss[slot])
        return carry

    lax.fori_loop(0, rpw // CHUNK, chunk_body, 0)
    for i in range(2):
        pltpu.make_async_copy(
            srows[i], scores_hbm.at[base + rpw - 2 + i], sem_ss[i]).wait()


  return _sc_scores_body


@functools.cache
def _sc_scores_kernel(nb):
    rpw = nb // NW
    mesh = plsc.VectorSubcoreMesh(
        core_axis_name="c", subcore_axis_name="s",
        num_cores=SC_CORES, num_subcores=SC_SUBCORES)
    return pl.kernel(
        _make_sc_scores_body(rpw),
        out_type=jax.ShapeDtypeStruct((nb, D), jnp.float32),
        mesh=mesh,
        scratch_types=[
            pltpu.VMEM((rpw, IDX_PAD), jnp.int32),
            pltpu.VMEM((rpw,), jnp.int32),
            pltpu.VMEM((CHUNK, D), jnp.float32),
            pltpu.VMEM((K, D), jnp.float32),
            pltpu.VMEM((K, D), jnp.float32),
            pltpu.VMEM((D,), jnp.float32),
            pltpu.VMEM((D,), jnp.float32),
            pltpu.SemaphoreType.DMA,
            pltpu.SemaphoreType.DMA,
            pltpu.SemaphoreType.DMA,
            pltpu.SemaphoreType.DMA,
            pltpu.SemaphoreType.DMA,
            pltpu.SemaphoreType.DMA,
        ],
    )


def _select_body(s_ref, x_ref, o_ref):
    bits = lax.bitcast_convert_type(s_ref[...], jnp.int32)
    rb = bits.shape[0]

    def round_body(t, prefix):
        cand = prefix | (1 << (30 - t))
        cnt = jnp.sum((bits >= cand).astype(jnp.int32), axis=1, keepdims=True)
        return jnp.where(cnt >= NUM_DROP, cand, prefix)

    prefix = lax.fori_loop(0, 31, round_body, jnp.zeros((rb, 1), jnp.int32))
    o_ref[...] = jnp.where(bits >= prefix, 0.0, x_ref[...])


def _select_mask(scores, x):
    rb = 1024
    nb = scores.shape[0]
    return pl.pallas_call(
        _select_body,
        grid=(nb // rb,),
        in_specs=[
            pl.BlockSpec((rb, D), lambda i: (i, 0)),
            pl.BlockSpec((rb, D), lambda i: (i, 0)),
        ],
        out_specs=pl.BlockSpec((rb, D), lambda i: (i, 0)),
        out_shape=jax.ShapeDtypeStruct((nb, D), jnp.float32),
    )(scores, x)


def kernel(x, y, weight_matrix, prev_output):
    y1 = y.astype(jnp.int32)
    p_pad = jnp.pad(prev_output, ((0, 0), (0, CPAD - C)),
                    constant_values=-jnp.inf)
    idx = _topk_indices(p_pad, y1.reshape(B, 1))
    scores = _sc_scores_kernel(B)(idx, y1, weight_matrix)
    return _select_mask(scores, x)
